# baseline (speedup 1.0000x reference)
import functools

import jax
import jax.numpy as jnp
from jax import lax
from jax.experimental import pallas as pl
from jax.experimental.pallas import tpu as pltpu
from jax.experimental.pallas import tpu_sc as plsc

_T = 16
_K = 3072
_N = 24000
_B = 4
_C = 6
_L = 16
_NV = _N // _L
_SEL = _K + 2 * _L
_SELV = _SEL // _L
_INF = 0x7F800000
_BIG = 0x7FFFFFFF


def _d2_body(x_ref, xt_ref, lab_ref, keys_ref):
    xb = x_ref[0]
    coords = xt_ref[0][:, 0:3]
    lab = lab_ref[0]

    tooth = lax.broadcasted_iota(jnp.int32, (_T, _N), 0)
    onehot = (lab == tooth).astype(jnp.float32)
    counts = jnp.sum(onehot, axis=1)
    sums = lax.dot_general(onehot, coords, (((1,), (0,)), ((), ())),
                           preferred_element_type=jnp.float32)
    cen = sums / jnp.maximum(counts, 1.0)[:, None]

    px = xb[0:1, :]
    py = xb[1:2, :]
    pz = xb[2:3, :]
    dx = cen[:, 0:1] - px
    dy = cen[:, 1:2] - py
    dz = cen[:, 2:3] - pz
    d2 = (dx * dx + dy * dy) + dz * dz
    keys_ref[0] = lax.bitcast_convert_type(d2, jnp.int32)


def _sc_body(keys_hbm, xt_hbm, idx_hbm, cent_hbm,
             keys_v, selk_a, seli_a, selk_b, seli_b, tie_v,
             hist_v, pref_v, gbuf, obuf, sem):
    wid = lax.axis_index("s") * 2 + lax.axis_index("c")

    def process_row(j, _):
        row = wid + 32 * j
        b = row // _T

        pltpu.sync_copy(keys_hbm.at[row], keys_v)

        def select_level(lvl, carry):
            pref, kth, n_less = carry
            shift = 24 - 8 * lvl

            def clear_body(i, c):
                hist_v[pl.ds(i * _L, _L)] = jnp.zeros((_L,), jnp.int32)
                return c
            lax.fori_loop(0, 256 // _L, clear_body, 0)

            def hist_body(v, c):
                kv = keys_v[pl.ds(v * _L, _L)]
                m = lax.shift_right_logical(kv, shift + 8) == pref
                digit = lax.shift_right_logical(kv, shift) & 255
                cnt, lm = plsc.scan_count(digit, mask=m)
                plsc.addupdate_scatter(hist_v, [digit], cnt,
                                       mask=jnp.logical_and(lm, m))
                return c
            lax.fori_loop(0, _NV, hist_body, 0)

            def beta_body(i, c):
                cum, beta = c
                h = hist_v[pl.ds(i * _L, _L)]
                inc = plsc.cumsum(h) + cum
                bins = lax.iota(jnp.int32, _L) + i * _L
                cand = jnp.where(inc > kth, bins, 256)
                return cum + jnp.sum(h), jnp.minimum(beta, jnp.min(cand))
            _, beta = lax.fori_loop(0, 256 // _L, beta_body,
                                    (jnp.int32(0), jnp.int32(256)))

            def nb_body(i, acc):
                h = hist_v[pl.ds(i * _L, _L)]
                bins = lax.iota(jnp.int32, _L) + i * _L
                return acc + jnp.sum(jnp.where(bins < beta, h, 0))
            nb = lax.fori_loop(0, 256 // _L, nb_body, jnp.int32(0))

            return pref * 256 + beta, kth - nb, n_less + nb

        pref, kth, n_less = lax.fori_loop(
            0, 4, select_level,
            (jnp.int32(0), jnp.int32(_K - 1), jnp.int32(0)))
        thr = pref
        needed = kth + 1

        def init_body(i, c):
            selk_a[pl.ds(i * _L, _L)] = jnp.full((_L,), _INF, jnp.int32)
            seli_a[pl.ds(i * _L, _L)] = jnp.full((_L,), _BIG, jnp.int32)
            tie_v[pl.ds(i * _L, _L)] = jnp.full((_L,), _BIG, jnp.int32)
            return c
        lax.fori_loop(0, _SELV, init_body, 0)

        def comp_body(v, carry):
            off, toff = carry
            kv = keys_v[pl.ds(v * _L, _L)]
            idxv = lax.iota(jnp.int32, _L) + v * _L
            m_lt = kv < thr
            plsc.store_compressed(selk_a.at[pl.ds(off, _L)], kv, mask=m_lt)
            plsc.store_compressed(seli_a.at[pl.ds(off, _L)], idxv, mask=m_lt)
            m_eq = kv == thr

            @pl.when(toff < needed)
            def _():
                plsc.store_compressed(tie_v.at[pl.ds(toff, _L)], idxv,
                                      mask=m_eq)

            off = off + jnp.sum(m_lt.astype(jnp.int32))
            toff = jnp.where(toff < needed,
                             toff + jnp.sum(m_eq.astype(jnp.int32)), toff)
            return off, toff
        lax.fori_loop(0, _NV, comp_body, (jnp.int32(0), jnp.int32(0)))

        def app_body(i, c):
            selk_a[pl.ds(n_less + i * _L, _L)] = jnp.full((_L,), 0, jnp.int32) + thr
            seli_a[pl.ds(n_less + i * _L, _L)] = tie_v[pl.ds(i * _L, _L)]
            return c
        lax.fori_loop(0, (needed + _L - 1) // _L, app_body, 0)

        def sort_pass(p, sk, si, dk, di):
            shift = 8 * p

            def clear_body(i, c):
                hist_v[pl.ds(i * _L, _L)] = jnp.zeros((_L,), jnp.int32)
                return c
            lax.fori_loop(0, 256 // _L, clear_body, 0)

            def hist_body(v, c):
                kv = sk[pl.ds(v * _L, _L)]
                digit = lax.shift_right_logical(kv, shift) & 255
                cnt, lm = plsc.scan_count(digit)
                plsc.addupdate_scatter(hist_v, [digit], cnt, mask=lm)
                return c
            lax.fori_loop(0, _SELV, hist_body, 0)

            def pf_body(i, cum):
                h = hist_v[pl.ds(i * _L, _L)]
                inc = plsc.cumsum(h)
                pref_v[pl.ds(i * _L, _L)] = inc - h + cum
                return cum + jnp.sum(h)
            lax.fori_loop(0, 256 // _L, pf_body, jnp.int32(0))

            def perm_body(v, c):
                kv = sk[pl.ds(v * _L, _L)]
                iv = si[pl.ds(v * _L, _L)]
                digit = lax.shift_right_logical(kv, shift) & 255
                cnt, lm = plsc.scan_count(digit)
                base = plsc.load_gather(pref_v, [digit])
                pos = base + cnt - 1
                plsc.store_scatter(dk, [pos], kv)
                plsc.store_scatter(di, [pos], iv)
                plsc.addupdate_scatter(pref_v, [digit], cnt, mask=lm)
                return c
            lax.fori_loop(0, _SELV, perm_body, 0)

        sort_pass(0, selk_a, seli_a, selk_b, seli_b)
        sort_pass(1, selk_b, seli_b, selk_a, seli_a)
        sort_pass(2, selk_a, seli_a, selk_b, seli_b)
        sort_pass(3, selk_b, seli_b, selk_a, seli_a)

        pltpu.sync_copy(seli_a.at[pl.ds(0, _K)], idx_hbm.at[row])

        pltpu.async_copy(xt_hbm.at[b].at[seli_a.at[pl.ds(0, _K)]],
                         gbuf, sem).wait()

        for c in range(_C):
            col = jnp.full((_L,), c, jnp.int32)

            if c < 3:
                def acc_body(i, acc, _col=col):
                    rows16 = lax.iota(jnp.int32, _L) + i * _L
                    return acc + plsc.load_gather(gbuf, [rows16, _col])
                acc = lax.fori_loop(0, _K // _L, acc_body,
                                    jnp.zeros((_L,), jnp.float32))
                mean = jnp.sum(acc) * (1.0 / _K)
            else:
                mean = jnp.float32(0.0)

            def deint_body(i, cc, _col=col, _m=mean):
                rows16 = lax.iota(jnp.int32, _L) + i * _L
                obuf[pl.ds(i * _L, _L)] = (
                    plsc.load_gather(gbuf, [rows16, _col]) - _m)
                return cc
            lax.fori_loop(0, _K // _L, deint_body, 0)

            pltpu.sync_copy(obuf, cent_hbm.at[row, c])
        return 0

    lax.fori_loop(0, 2, process_row, 0)


@functools.partial(jax.jit, static_argnums=())
def _sc_stage(keys, x):
    return pl.kernel(
        _sc_body,
        out_type=[
            jax.ShapeDtypeStruct((_B * _T, _K), jnp.int32),
            jax.ShapeDtypeStruct((_B * _T, _C, _K), jnp.float32),
        ],
        mesh=plsc.VectorSubcoreMesh(core_axis_name="c", subcore_axis_name="s"),
        compiler_params=pltpu.CompilerParams(needs_layout_passes=False,
                                             use_tc_tiling_on_sc=False),
        scratch_types=[
            pltpu.VMEM((_N,), jnp.int32),
            pltpu.VMEM((_SEL,), jnp.int32),
            pltpu.VMEM((_SEL,), jnp.int32),
            pltpu.VMEM((_SEL,), jnp.int32),
            pltpu.VMEM((_SEL,), jnp.int32),
            pltpu.VMEM((_SEL,), jnp.int32),
            pltpu.VMEM((256,), jnp.int32),
            pltpu.VMEM((256,), jnp.int32),
            pltpu.VMEM((_K, 8), jnp.float32),
            pltpu.VMEM((_K,), jnp.float32),
            pltpu.SemaphoreType.DMA,
        ],
    )(keys, x)


def kernel(inputs_0, inputs_1):
    x = inputs_0
    labels = inputs_1
    B, C, N = x.shape

    xt = jnp.pad(jnp.transpose(x, (0, 2, 1)), ((0, 0), (0, 0), (0, 2)))

    keys = pl.pallas_call(
        _d2_body,
        grid=(B,),
        in_specs=[
            pl.BlockSpec((1, C, N), lambda i: (i, 0, 0)),
            pl.BlockSpec((1, N, 8), lambda i: (i, 0, 0)),
            pl.BlockSpec((1, 1, N), lambda i: (i, 0, 0)),
        ],
        out_specs=pl.BlockSpec((1, _T, N), lambda i: (i, 0, 0)),
        out_shape=jax.ShapeDtypeStruct((B, _T, N), jnp.int32),
    )(x, xt, labels)
    keys = keys.reshape(B * _T, N)

    nn_flat, centered = _sc_stage(keys, xt)
    return centered, nn_flat.reshape(B, _T, _K)


# speedup vs baseline: 1.4847x; 1.4847x over previous
import functools

import jax
import jax.numpy as jnp
from jax import lax
from jax.experimental import pallas as pl
from jax.experimental.pallas import tpu as pltpu
from jax.experimental.pallas import tpu_sc as plsc

_T = 16
_K = 3072
_N = 24000
_B = 4
_C = 6
_L = 16
_NV = _N // _L
_SEL = _K + 2 * _L
_SELV = _SEL // _L
_HBINS = 4096
_INF = 0x7F800000
_BIG = 0x7FFFFFFF


def _d2_body(x_ref, xt_ref, lab_ref, keys_ref):
    xb = x_ref[0]
    coords = xt_ref[0][:, 0:3]
    lab = lab_ref[0]

    tooth = lax.broadcasted_iota(jnp.int32, (_T, _N), 0)
    onehot = (lab == tooth).astype(jnp.float32)
    counts = jnp.sum(onehot, axis=1)
    sums = lax.dot_general(onehot, coords, (((1,), (0,)), ((), ())),
                           preferred_element_type=jnp.float32)
    cen = sums / jnp.maximum(counts, 1.0)[:, None]

    px = xb[0:1, :]
    py = xb[1:2, :]
    pz = xb[2:3, :]
    dx = cen[:, 0:1] - px
    dy = cen[:, 1:2] - py
    dz = cen[:, 2:3] - pz
    d2 = (dx * dx + dy * dy) + dz * dz
    keys_ref[0] = lax.bitcast_convert_type(d2, jnp.int32)


def _sc_body(keys_hbm, xt_hbm, idx_hbm, cent_hbm,
             keys_v, selk_a, seli_a, selk_b, seli_b, tie_v,
             hist_v, pref_v, gbuf, obuf, sem):
    wid = lax.axis_index("s") * 2 + lax.axis_index("c")

    def process_row(j, _):
        row = wid + 32 * j
        b = row // _T

        pltpu.sync_copy(keys_hbm.at[row], keys_v)

        def init_body(i, c):
            selk_a[pl.ds(i * _L, _L)] = jnp.full((_L,), _INF, jnp.int32)
            seli_a[pl.ds(i * _L, _L)] = jnp.full((_L,), _BIG, jnp.int32)
            tie_v[pl.ds(i * _L, _L)] = jnp.full((_L,), _BIG, jnp.int32)
            return c
        lax.fori_loop(0, _SELV, init_body, 0)

        def clear12_body(i, c):
            hist_v[pl.ds(i * _L, _L)] = jnp.zeros((_L,), jnp.int32)
            return c
        lax.fori_loop(0, _HBINS // _L, clear12_body, 0)

        def hist12_body(v, c):
            kv = keys_v[pl.ds(v * _L, _L)]
            digit = lax.shift_right_logical(kv, 20)
            cnt, lm = plsc.scan_count(digit)
            plsc.addupdate_scatter(hist_v, [digit], cnt, mask=lm)
            return c
        lax.fori_loop(0, _NV, hist12_body, 0)

        def beta12_body(i, c):
            cum, beta1, nb1, m1 = c
            h = hist_v[pl.ds(i * _L, _L)]
            inc = plsc.cumsum(h) + cum
            bins = lax.iota(jnp.int32, _L) + i * _L
            cand = jnp.where(inc > _K - 1, bins, _HBINS)
            bmin = jnp.min(cand)
            hit = jnp.logical_and(beta1 == _HBINS, bmin < _HBINS)
            nb_here = cum + jnp.sum(jnp.where(bins < bmin, h, 0))
            m_here = jnp.sum(jnp.where(bins == bmin, h, 0))
            nb1 = jnp.where(hit, nb_here, nb1)
            m1 = jnp.where(hit, m_here, m1)
            return (cum + jnp.sum(h), jnp.minimum(beta1, bmin), nb1, m1)
        _, beta1, nb1, m1 = lax.fori_loop(
            0, _HBINS // _L, beta12_body,
            (jnp.int32(0), jnp.int32(_HBINS), jnp.int32(0), jnp.int32(0)))

        @pl.when(m1 <= _SEL - _L)
        def _():
            def fp_scan2(v, carry):
                off, coff = carry
                kv = keys_v[pl.ds(v * _L, _L)]
                idxv = lax.iota(jnp.int32, _L) + v * _L
                top12 = lax.shift_right_logical(kv, 20)
                m_lt = top12 < beta1
                plsc.store_compressed(selk_a.at[pl.ds(off, _L)], kv,
                                      mask=m_lt)
                plsc.store_compressed(seli_a.at[pl.ds(off, _L)], idxv,
                                      mask=m_lt)
                m_eq = top12 == beta1
                plsc.store_compressed(selk_b.at[pl.ds(coff, _L)], kv,
                                      mask=m_eq)
                plsc.store_compressed(seli_b.at[pl.ds(coff, _L)], idxv,
                                      mask=m_eq)
                return (off + jnp.sum(m_lt.astype(jnp.int32)),
                        coff + jnp.sum(m_eq.astype(jnp.int32)))
            lax.fori_loop(0, _NV, fp_scan2, (jnp.int32(0), jnp.int32(0)))

            mv = (m1 + _L - 1) // _L

            pref_full = beta1
            kth_c = jnp.int32(_K - 1) - nb1
            n_less = nb1
            for shift, nbits in ((12, 8), (4, 8), (0, 4)):
                nchunk = (1 << nbits) // _L

                def cl_body(i, c):
                    hist_v[pl.ds(i * _L, _L)] = jnp.zeros((_L,), jnp.int32)
                    return c
                lax.fori_loop(0, nchunk, cl_body, 0)

                def ch_body(v, c, _s=shift, _n=nbits, _p=pref_full, _m1=m1):
                    kv = selk_b[pl.ds(v * _L, _L)]
                    rows = lax.iota(jnp.int32, _L) + v * _L
                    m = jnp.logical_and(
                        rows < _m1,
                        lax.shift_right_logical(kv, _s + _n) == _p)
                    digit = lax.shift_right_logical(kv, _s) & ((1 << _n) - 1)
                    cnt, lm = plsc.scan_count(digit, mask=m)
                    plsc.addupdate_scatter(hist_v, [digit], cnt,
                                           mask=jnp.logical_and(lm, m))
                    return c
                lax.fori_loop(0, mv, ch_body, 0)

                def cb_body(i, c, _n=nbits, _k=kth_c):
                    cum, beta, nb = c
                    h = hist_v[pl.ds(i * _L, _L)]
                    inc = plsc.cumsum(h) + cum
                    bins = lax.iota(jnp.int32, _L) + i * _L
                    cand = jnp.where(inc > _k, bins, 1 << _n)
                    bmin = jnp.min(cand)
                    hit = jnp.logical_and(beta == (1 << _n), bmin < (1 << _n))
                    nb_here = cum + jnp.sum(jnp.where(bins < bmin, h, 0))
                    nb = jnp.where(hit, nb_here, nb)
                    return (cum + jnp.sum(h), jnp.minimum(beta, bmin), nb)
                _, beta_l, nb_l = lax.fori_loop(
                    0, nchunk, cb_body,
                    (jnp.int32(0), jnp.int32(1 << nbits), jnp.int32(0)))

                pref_full = pref_full * (1 << nbits) + beta_l
                kth_c = kth_c - nb_l
                n_less = n_less + nb_l

            thr = pref_full
            needed = kth_c + 1

            def fp_comp(v, carry, _m1=m1):
                off, toff = carry
                kv = selk_b[pl.ds(v * _L, _L)]
                iv = seli_b[pl.ds(v * _L, _L)]
                rows = lax.iota(jnp.int32, _L) + v * _L
                mval = rows < _m1
                m_lt = jnp.logical_and(kv < thr, mval)
                plsc.store_compressed(selk_a.at[pl.ds(off, _L)], kv,
                                      mask=m_lt)
                plsc.store_compressed(seli_a.at[pl.ds(off, _L)], iv,
                                      mask=m_lt)
                m_eq = jnp.logical_and(kv == thr, mval)

                @pl.when(toff < needed)
                def _():
                    plsc.store_compressed(tie_v.at[pl.ds(toff, _L)], iv,
                                          mask=m_eq)

                off = off + jnp.sum(m_lt.astype(jnp.int32))
                toff = jnp.where(toff < needed,
                                 toff + jnp.sum(m_eq.astype(jnp.int32)), toff)
                return off, toff
            lax.fori_loop(0, mv, fp_comp, (nb1, jnp.int32(0)))

            def fp_app(i, c):
                selk_a[pl.ds(n_less + i * _L, _L)] = (
                    jnp.full((_L,), 0, jnp.int32) + thr)
                seli_a[pl.ds(n_less + i * _L, _L)] = tie_v[pl.ds(i * _L, _L)]
                return c
            lax.fori_loop(0, (needed + _L - 1) // _L, fp_app, 0)

        @pl.when(m1 > _SEL - _L)
        def _():
            def select_level(lvl, carry):
                pref, kth, nl = carry
                shift = 24 - 8 * lvl

                def clear_body(i, c):
                    hist_v[pl.ds(i * _L, _L)] = jnp.zeros((_L,), jnp.int32)
                    return c
                lax.fori_loop(0, 256 // _L, clear_body, 0)

                def hist_body(v, c):
                    kv = keys_v[pl.ds(v * _L, _L)]
                    m = lax.shift_right_logical(kv, shift + 8) == pref
                    digit = lax.shift_right_logical(kv, shift) & 255
                    cnt, lm = plsc.scan_count(digit, mask=m)
                    plsc.addupdate_scatter(hist_v, [digit], cnt,
                                           mask=jnp.logical_and(lm, m))
                    return c
                lax.fori_loop(0, _NV, hist_body, 0)

                def beta_body(i, c):
                    cum, beta = c
                    h = hist_v[pl.ds(i * _L, _L)]
                    inc = plsc.cumsum(h) + cum
                    bins = lax.iota(jnp.int32, _L) + i * _L
                    cand = jnp.where(inc > kth, bins, 256)
                    return cum + jnp.sum(h), jnp.minimum(beta, jnp.min(cand))
                _, beta = lax.fori_loop(0, 256 // _L, beta_body,
                                        (jnp.int32(0), jnp.int32(256)))

                def nb_body(i, acc):
                    h = hist_v[pl.ds(i * _L, _L)]
                    bins = lax.iota(jnp.int32, _L) + i * _L
                    return acc + jnp.sum(jnp.where(bins < beta, h, 0))
                nb = lax.fori_loop(0, 256 // _L, nb_body, jnp.int32(0))

                return pref * 256 + beta, kth - nb, nl + nb

            pref, kth, n_less = lax.fori_loop(
                0, 4, select_level,
                (jnp.int32(0), jnp.int32(_K - 1), jnp.int32(0)))
            thr = pref
            needed = kth + 1

            def comp_body(v, carry):
                off, toff = carry
                kv = keys_v[pl.ds(v * _L, _L)]
                idxv = lax.iota(jnp.int32, _L) + v * _L
                m_lt = kv < thr
                plsc.store_compressed(selk_a.at[pl.ds(off, _L)], kv,
                                      mask=m_lt)
                plsc.store_compressed(seli_a.at[pl.ds(off, _L)], idxv,
                                      mask=m_lt)
                m_eq = kv == thr

                @pl.when(toff < needed)
                def _():
                    plsc.store_compressed(tie_v.at[pl.ds(toff, _L)], idxv,
                                          mask=m_eq)

                off = off + jnp.sum(m_lt.astype(jnp.int32))
                toff = jnp.where(toff < needed,
                                 toff + jnp.sum(m_eq.astype(jnp.int32)), toff)
                return off, toff
            lax.fori_loop(0, _NV, comp_body, (jnp.int32(0), jnp.int32(0)))

            def app_body(i, c):
                selk_a[pl.ds(n_less + i * _L, _L)] = (
                    jnp.full((_L,), 0, jnp.int32) + thr)
                seli_a[pl.ds(n_less + i * _L, _L)] = tie_v[pl.ds(i * _L, _L)]
                return c
            lax.fori_loop(0, (needed + _L - 1) // _L, app_body, 0)

        def sort_pass(p, sk, si, dk, di):
            shift = 8 * p

            def clear_body(i, c):
                hist_v[pl.ds(i * _L, _L)] = jnp.zeros((_L,), jnp.int32)
                return c
            lax.fori_loop(0, 256 // _L, clear_body, 0)

            def hist_body(v, c):
                kv = sk[pl.ds(v * _L, _L)]
                digit = lax.shift_right_logical(kv, shift) & 255
                cnt, lm = plsc.scan_count(digit)
                plsc.addupdate_scatter(hist_v, [digit], cnt, mask=lm)
                return c
            lax.fori_loop(0, _SELV, hist_body, 0)

            def pf_body(i, cum):
                h = hist_v[pl.ds(i * _L, _L)]
                inc = plsc.cumsum(h)
                pref_v[pl.ds(i * _L, _L)] = inc - h + cum
                return cum + jnp.sum(h)
            lax.fori_loop(0, 256 // _L, pf_body, jnp.int32(0))

            def perm_body(v, c):
                kv = sk[pl.ds(v * _L, _L)]
                iv = si[pl.ds(v * _L, _L)]
                digit = lax.shift_right_logical(kv, shift) & 255
                cnt, lm = plsc.scan_count(digit)
                base = plsc.load_gather(pref_v, [digit])
                pos = base + cnt - 1
                plsc.store_scatter(dk, [pos], kv)
                plsc.store_scatter(di, [pos], iv)
                plsc.addupdate_scatter(pref_v, [digit], cnt, mask=lm)
                return c
            lax.fori_loop(0, _SELV, perm_body, 0)

        sort_pass(0, selk_a, seli_a, selk_b, seli_b)
        sort_pass(1, selk_b, seli_b, selk_a, seli_a)
        sort_pass(2, selk_a, seli_a, selk_b, seli_b)
        sort_pass(3, selk_b, seli_b, selk_a, seli_a)

        pltpu.sync_copy(seli_a.at[pl.ds(0, _K)], idx_hbm.at[row])

        pltpu.async_copy(xt_hbm.at[b].at[seli_a.at[pl.ds(0, _K)]],
                         gbuf, sem).wait()

        for c in range(_C):
            col = jnp.full((_L,), c, jnp.int32)

            if c < 3:
                def acc_body(i, acc, _col=col):
                    rows16 = lax.iota(jnp.int32, _L) + i * _L
                    return acc + plsc.load_gather(gbuf, [rows16, _col])
                acc = lax.fori_loop(0, _K // _L, acc_body,
                                    jnp.zeros((_L,), jnp.float32))
                mean = jnp.sum(acc) * (1.0 / _K)
            else:
                mean = jnp.float32(0.0)

            def deint_body(i, cc, _col=col, _m=mean):
                rows16 = lax.iota(jnp.int32, _L) + i * _L
                obuf[pl.ds(i * _L, _L)] = (
                    plsc.load_gather(gbuf, [rows16, _col]) - _m)
                return cc
            lax.fori_loop(0, _K // _L, deint_body, 0)

            pltpu.sync_copy(obuf, cent_hbm.at[row, c])
        return 0

    lax.fori_loop(0, 2, process_row, 0)


@functools.partial(jax.jit, static_argnums=())
def _sc_stage(keys, x):
    return pl.kernel(
        _sc_body,
        out_type=[
            jax.ShapeDtypeStruct((_B * _T, _K), jnp.int32),
            jax.ShapeDtypeStruct((_B * _T, _C, _K), jnp.float32),
        ],
        mesh=plsc.VectorSubcoreMesh(core_axis_name="c", subcore_axis_name="s"),
        compiler_params=pltpu.CompilerParams(needs_layout_passes=False,
                                             use_tc_tiling_on_sc=False),
        scratch_types=[
            pltpu.VMEM((_N,), jnp.int32),
            pltpu.VMEM((_SEL,), jnp.int32),
            pltpu.VMEM((_SEL,), jnp.int32),
            pltpu.VMEM((_SEL,), jnp.int32),
            pltpu.VMEM((_SEL,), jnp.int32),
            pltpu.VMEM((_SEL,), jnp.int32),
            pltpu.VMEM((_HBINS,), jnp.int32),
            pltpu.VMEM((256,), jnp.int32),
            pltpu.VMEM((_K, 8), jnp.float32),
            pltpu.VMEM((_K,), jnp.float32),
            pltpu.SemaphoreType.DMA,
        ],
    )(keys, x)


def kernel(inputs_0, inputs_1):
    x = inputs_0
    labels = inputs_1
    B, C, N = x.shape

    xt = jnp.pad(jnp.transpose(x, (0, 2, 1)), ((0, 0), (0, 0), (0, 2)))

    keys = pl.pallas_call(
        _d2_body,
        grid=(B,),
        in_specs=[
            pl.BlockSpec((1, C, N), lambda i: (i, 0, 0)),
            pl.BlockSpec((1, N, 8), lambda i: (i, 0, 0)),
            pl.BlockSpec((1, 1, N), lambda i: (i, 0, 0)),
        ],
        out_specs=pl.BlockSpec((1, _T, N), lambda i: (i, 0, 0)),
        out_shape=jax.ShapeDtypeStruct((B, _T, N), jnp.int32),
    )(x, xt, labels)
    keys = keys.reshape(B * _T, N)

    nn_flat, centered = _sc_stage(keys, xt)
    return centered, nn_flat.reshape(B, _T, _K)


# speedup vs baseline: 1.6570x; 1.1161x over previous
import functools

import jax
import jax.numpy as jnp
from jax import lax
from jax.experimental import pallas as pl
from jax.experimental.pallas import tpu as pltpu
from jax.experimental.pallas import tpu_sc as plsc

_T = 16
_K = 3072
_N = 24000
_B = 4
_C = 6
_L = 16
_NV = _N // _L
_SEL = _K + 2 * _L
_SELV = _SEL // _L
_HBINS = 4096
_INF = 0x7F800000
_BIG = 0x7FFFFFFF


def _d2_body(x_ref, coords_ref, lab_ref, keys_ref):
    xb = x_ref[0]
    coords = coords_ref[0]
    lab = lab_ref[0]

    tooth = lax.broadcasted_iota(jnp.int32, (_T, _N), 0)
    onehot = (lab == tooth).astype(jnp.float32)
    counts = jnp.sum(onehot, axis=1)
    sums = lax.dot_general(onehot, coords, (((1,), (0,)), ((), ())),
                           preferred_element_type=jnp.float32)
    cen = sums / jnp.maximum(counts, 1.0)[:, None]

    px = xb[0:1, :]
    py = xb[1:2, :]
    pz = xb[2:3, :]
    dx = cen[:, 0:1] - px
    dy = cen[:, 1:2] - py
    dz = cen[:, 2:3] - pz
    d2 = (dx * dx + dy * dy) + dz * dz
    keys_ref[0] = lax.bitcast_convert_type(d2, jnp.int32)


def _sc_body(keys_hbm, x_hbm, idx_hbm, cent_hbm,
             keys_v, selk_a, seli_a, selk_b, seli_b, tie_v,
             hist_v, pref_v, gbuf, obuf, sem):
    wid = lax.axis_index("s") * 2 + lax.axis_index("c")

    def process_row(j, _):
        row = wid + 32 * j
        b = row // _T

        pltpu.sync_copy(keys_hbm.at[row], keys_v)

        def init_body(i, c):
            selk_a[pl.ds(i * _L, _L)] = jnp.full((_L,), _INF, jnp.int32)
            seli_a[pl.ds(i * _L, _L)] = jnp.full((_L,), _BIG, jnp.int32)
            tie_v[pl.ds(i * _L, _L)] = jnp.full((_L,), _BIG, jnp.int32)
            return c
        lax.fori_loop(0, _SELV, init_body, 0)

        def clear12_body(i, c):
            hist_v[pl.ds(i * _L, _L)] = jnp.zeros((_L,), jnp.int32)
            return c
        lax.fori_loop(0, _HBINS // _L, clear12_body, 0)

        def hist12_body(v, c):
            kv = keys_v[pl.ds(v * _L, _L)]
            digit = lax.shift_right_logical(kv, 20)
            cnt, lm = plsc.scan_count(digit)
            plsc.addupdate_scatter(hist_v, [digit], cnt, mask=lm)
            return c
        lax.fori_loop(0, _NV, hist12_body, 0)

        def beta12_body(i, c):
            cum, beta1, nb1, m1 = c
            h = hist_v[pl.ds(i * _L, _L)]
            inc = plsc.cumsum(h) + cum
            bins = lax.iota(jnp.int32, _L) + i * _L
            cand = jnp.where(inc > _K - 1, bins, _HBINS)
            bmin = jnp.min(cand)
            hit = jnp.logical_and(beta1 == _HBINS, bmin < _HBINS)
            nb_here = cum + jnp.sum(jnp.where(bins < bmin, h, 0))
            m_here = jnp.sum(jnp.where(bins == bmin, h, 0))
            nb1 = jnp.where(hit, nb_here, nb1)
            m1 = jnp.where(hit, m_here, m1)
            return (cum + jnp.sum(h), jnp.minimum(beta1, bmin), nb1, m1)
        _, beta1, nb1, m1 = lax.fori_loop(
            0, _HBINS // _L, beta12_body,
            (jnp.int32(0), jnp.int32(_HBINS), jnp.int32(0), jnp.int32(0)))

        @pl.when(m1 <= _SEL - _L)
        def _():
            def fp_scan2(v, carry):
                off, coff = carry
                kv = keys_v[pl.ds(v * _L, _L)]
                idxv = lax.iota(jnp.int32, _L) + v * _L
                top12 = lax.shift_right_logical(kv, 20)
                m_lt = top12 < beta1
                plsc.store_compressed(selk_a.at[pl.ds(off, _L)], kv,
                                      mask=m_lt)
                plsc.store_compressed(seli_a.at[pl.ds(off, _L)], idxv,
                                      mask=m_lt)
                m_eq = top12 == beta1
                plsc.store_compressed(selk_b.at[pl.ds(coff, _L)], kv,
                                      mask=m_eq)
                plsc.store_compressed(seli_b.at[pl.ds(coff, _L)], idxv,
                                      mask=m_eq)
                return (off + jnp.sum(m_lt.astype(jnp.int32)),
                        coff + jnp.sum(m_eq.astype(jnp.int32)))
            lax.fori_loop(0, _NV, fp_scan2, (jnp.int32(0), jnp.int32(0)))

            mv = (m1 + _L - 1) // _L

            pref_full = beta1
            kth_c = jnp.int32(_K - 1) - nb1
            n_less = nb1
            for shift, nbits in ((12, 8), (4, 8), (0, 4)):
                nchunk = (1 << nbits) // _L

                def cl_body(i, c):
                    hist_v[pl.ds(i * _L, _L)] = jnp.zeros((_L,), jnp.int32)
                    return c
                lax.fori_loop(0, nchunk, cl_body, 0)

                def ch_body(v, c, _s=shift, _n=nbits, _p=pref_full, _m1=m1):
                    kv = selk_b[pl.ds(v * _L, _L)]
                    rows = lax.iota(jnp.int32, _L) + v * _L
                    m = jnp.logical_and(
                        rows < _m1,
                        lax.shift_right_logical(kv, _s + _n) == _p)
                    digit = lax.shift_right_logical(kv, _s) & ((1 << _n) - 1)
                    cnt, lm = plsc.scan_count(digit, mask=m)
                    plsc.addupdate_scatter(hist_v, [digit], cnt,
                                           mask=jnp.logical_and(lm, m))
                    return c
                lax.fori_loop(0, mv, ch_body, 0)

                def cb_body(i, c, _n=nbits, _k=kth_c):
                    cum, beta, nb = c
                    h = hist_v[pl.ds(i * _L, _L)]
                    inc = plsc.cumsum(h) + cum
                    bins = lax.iota(jnp.int32, _L) + i * _L
                    cand = jnp.where(inc > _k, bins, 1 << _n)
                    bmin = jnp.min(cand)
                    hit = jnp.logical_and(beta == (1 << _n), bmin < (1 << _n))
                    nb_here = cum + jnp.sum(jnp.where(bins < bmin, h, 0))
                    nb = jnp.where(hit, nb_here, nb)
                    return (cum + jnp.sum(h), jnp.minimum(beta, bmin), nb)
                _, beta_l, nb_l = lax.fori_loop(
                    0, nchunk, cb_body,
                    (jnp.int32(0), jnp.int32(1 << nbits), jnp.int32(0)))

                pref_full = pref_full * (1 << nbits) + beta_l
                kth_c = kth_c - nb_l
                n_less = n_less + nb_l

            thr = pref_full
            needed = kth_c + 1

            def fp_comp(v, carry, _m1=m1):
                off, toff = carry
                kv = selk_b[pl.ds(v * _L, _L)]
                iv = seli_b[pl.ds(v * _L, _L)]
                rows = lax.iota(jnp.int32, _L) + v * _L
                mval = rows < _m1
                m_lt = jnp.logical_and(kv < thr, mval)
                plsc.store_compressed(selk_a.at[pl.ds(off, _L)], kv,
                                      mask=m_lt)
                plsc.store_compressed(seli_a.at[pl.ds(off, _L)], iv,
                                      mask=m_lt)
                m_eq = jnp.logical_and(kv == thr, mval)

                @pl.when(toff < needed)
                def _():
                    plsc.store_compressed(tie_v.at[pl.ds(toff, _L)], iv,
                                          mask=m_eq)

                off = off + jnp.sum(m_lt.astype(jnp.int32))
                toff = jnp.where(toff < needed,
                                 toff + jnp.sum(m_eq.astype(jnp.int32)), toff)
                return off, toff
            lax.fori_loop(0, mv, fp_comp, (nb1, jnp.int32(0)))

            def fp_app(i, c):
                selk_a[pl.ds(n_less + i * _L, _L)] = (
                    jnp.full((_L,), 0, jnp.int32) + thr)
                seli_a[pl.ds(n_less + i * _L, _L)] = tie_v[pl.ds(i * _L, _L)]
                return c
            lax.fori_loop(0, (needed + _L - 1) // _L, fp_app, 0)

        @pl.when(m1 > _SEL - _L)
        def _():
            def select_level(lvl, carry):
                pref, kth, nl = carry
                shift = 24 - 8 * lvl

                def clear_body(i, c):
                    hist_v[pl.ds(i * _L, _L)] = jnp.zeros((_L,), jnp.int32)
                    return c
                lax.fori_loop(0, 256 // _L, clear_body, 0)

                def hist_body(v, c):
                    kv = keys_v[pl.ds(v * _L, _L)]
                    m = lax.shift_right_logical(kv, shift + 8) == pref
                    digit = lax.shift_right_logical(kv, shift) & 255
                    cnt, lm = plsc.scan_count(digit, mask=m)
                    plsc.addupdate_scatter(hist_v, [digit], cnt,
                                           mask=jnp.logical_and(lm, m))
                    return c
                lax.fori_loop(0, _NV, hist_body, 0)

                def beta_body(i, c):
                    cum, beta = c
                    h = hist_v[pl.ds(i * _L, _L)]
                    inc = plsc.cumsum(h) + cum
                    bins = lax.iota(jnp.int32, _L) + i * _L
                    cand = jnp.where(inc > kth, bins, 256)
                    return cum + jnp.sum(h), jnp.minimum(beta, jnp.min(cand))
                _, beta = lax.fori_loop(0, 256 // _L, beta_body,
                                        (jnp.int32(0), jnp.int32(256)))

                def nb_body(i, acc):
                    h = hist_v[pl.ds(i * _L, _L)]
                    bins = lax.iota(jnp.int32, _L) + i * _L
                    return acc + jnp.sum(jnp.where(bins < beta, h, 0))
                nb = lax.fori_loop(0, 256 // _L, nb_body, jnp.int32(0))

                return pref * 256 + beta, kth - nb, nl + nb

            pref, kth, n_less = lax.fori_loop(
                0, 4, select_level,
                (jnp.int32(0), jnp.int32(_K - 1), jnp.int32(0)))
            thr = pref
            needed = kth + 1

            def comp_body(v, carry):
                off, toff = carry
                kv = keys_v[pl.ds(v * _L, _L)]
                idxv = lax.iota(jnp.int32, _L) + v * _L
                m_lt = kv < thr
                plsc.store_compressed(selk_a.at[pl.ds(off, _L)], kv,
                                      mask=m_lt)
                plsc.store_compressed(seli_a.at[pl.ds(off, _L)], idxv,
                                      mask=m_lt)
                m_eq = kv == thr

                @pl.when(toff < needed)
                def _():
                    plsc.store_compressed(tie_v.at[pl.ds(toff, _L)], idxv,
                                          mask=m_eq)

                off = off + jnp.sum(m_lt.astype(jnp.int32))
                toff = jnp.where(toff < needed,
                                 toff + jnp.sum(m_eq.astype(jnp.int32)), toff)
                return off, toff
            lax.fori_loop(0, _NV, comp_body, (jnp.int32(0), jnp.int32(0)))

            def app_body(i, c):
                selk_a[pl.ds(n_less + i * _L, _L)] = (
                    jnp.full((_L,), 0, jnp.int32) + thr)
                seli_a[pl.ds(n_less + i * _L, _L)] = tie_v[pl.ds(i * _L, _L)]
                return c
            lax.fori_loop(0, (needed + _L - 1) // _L, app_body, 0)

        def sort_pass(p, sk, si, dk, di):
            shift = 8 * p

            def clear_body(i, c):
                hist_v[pl.ds(i * _L, _L)] = jnp.zeros((_L,), jnp.int32)
                return c
            lax.fori_loop(0, 256 // _L, clear_body, 0)

            def hist_body(v, c):
                kv = sk[pl.ds(v * _L, _L)]
                digit = lax.shift_right_logical(kv, shift) & 255
                cnt, lm = plsc.scan_count(digit)
                plsc.addupdate_scatter(hist_v, [digit], cnt, mask=lm)
                return c
            lax.fori_loop(0, _SELV, hist_body, 0)

            def pf_body(i, cum):
                h = hist_v[pl.ds(i * _L, _L)]
                inc = plsc.cumsum(h)
                pref_v[pl.ds(i * _L, _L)] = inc - h + cum
                return cum + jnp.sum(h)
            lax.fori_loop(0, 256 // _L, pf_body, jnp.int32(0))

            def perm_body(v, c):
                kv = sk[pl.ds(v * _L, _L)]
                iv = si[pl.ds(v * _L, _L)]
                digit = lax.shift_right_logical(kv, shift) & 255
                cnt, lm = plsc.scan_count(digit)
                base = plsc.load_gather(pref_v, [digit])
                pos = base + cnt - 1
                plsc.store_scatter(dk, [pos], kv)
                plsc.store_scatter(di, [pos], iv)
                plsc.addupdate_scatter(pref_v, [digit], cnt, mask=lm)
                return c
            lax.fori_loop(0, _SELV, perm_body, 0)

        sort_pass(0, selk_a, seli_a, selk_b, seli_b)
        sort_pass(1, selk_b, seli_b, selk_a, seli_a)
        sort_pass(2, selk_a, seli_a, selk_b, seli_b)
        sort_pass(3, selk_b, seli_b, selk_a, seli_a)

        pltpu.sync_copy(seli_a.at[pl.ds(0, _K)], idx_hbm.at[row])

        copies = [
            pltpu.async_copy(
                x_hbm.at[b].at[c].at[seli_a.at[pl.ds(0, _K)]],
                gbuf.at[c], sem)
            for c in range(_C)
        ]
        for cp in copies:
            cp.wait()

        for c in range(3):
            ch = gbuf.at[c]

            def acc_body(i, acc, _ch=ch):
                return acc + _ch[pl.ds(i * _L, _L)]
            acc = lax.fori_loop(0, _K // _L, acc_body,
                                jnp.zeros((_L,), jnp.float32))
            mean = jnp.sum(acc) * (1.0 / _K)

            def cen_body(i, cc, _ch=ch, _m=mean):
                obuf[pl.ds(i * _L, _L)] = _ch[pl.ds(i * _L, _L)] - _m
                return cc
            lax.fori_loop(0, _K // _L, cen_body, 0)

            pltpu.sync_copy(obuf, cent_hbm.at[row, c])

        for c in range(3, _C):
            pltpu.sync_copy(gbuf.at[c], cent_hbm.at[row, c])
        return 0

    lax.fori_loop(0, 2, process_row, 0)


@functools.partial(jax.jit, static_argnums=())
def _sc_stage(keys, x):
    return pl.kernel(
        _sc_body,
        out_type=[
            jax.ShapeDtypeStruct((_B * _T, _K), jnp.int32),
            jax.ShapeDtypeStruct((_B * _T, _C, _K), jnp.float32),
        ],
        mesh=plsc.VectorSubcoreMesh(core_axis_name="c", subcore_axis_name="s"),
        compiler_params=pltpu.CompilerParams(needs_layout_passes=False,
                                             use_tc_tiling_on_sc=False),
        scratch_types=[
            pltpu.VMEM((_N,), jnp.int32),
            pltpu.VMEM((_SEL,), jnp.int32),
            pltpu.VMEM((_SEL,), jnp.int32),
            pltpu.VMEM((_SEL,), jnp.int32),
            pltpu.VMEM((_SEL,), jnp.int32),
            pltpu.VMEM((_SEL,), jnp.int32),
            pltpu.VMEM((_HBINS,), jnp.int32),
            pltpu.VMEM((256,), jnp.int32),
            pltpu.VMEM((_C, _K), jnp.float32),
            pltpu.VMEM((_K,), jnp.float32),
            pltpu.SemaphoreType.DMA,
        ],
    )(keys, x)


def kernel(inputs_0, inputs_1):
    x = inputs_0
    labels = inputs_1
    B, C, N = x.shape

    coords = jnp.transpose(x[:, :3, :], (0, 2, 1))

    keys = pl.pallas_call(
        _d2_body,
        grid=(B,),
        in_specs=[
            pl.BlockSpec((1, C, N), lambda i: (i, 0, 0)),
            pl.BlockSpec((1, N, 3), lambda i: (i, 0, 0)),
            pl.BlockSpec((1, 1, N), lambda i: (i, 0, 0)),
        ],
        out_specs=pl.BlockSpec((1, _T, N), lambda i: (i, 0, 0)),
        out_shape=jax.ShapeDtypeStruct((B, _T, N), jnp.int32),
    )(x, coords, labels)
    keys = keys.reshape(B * _T, N)

    nn_flat, centered = _sc_stage(keys, x)
    return centered, nn_flat.reshape(B, _T, _K)


# speedup vs baseline: 1.8558x; 1.1199x over previous
import functools

import jax
import jax.numpy as jnp
from jax import lax
from jax.experimental import pallas as pl
from jax.experimental.pallas import tpu as pltpu
from jax.experimental.pallas import tpu_sc as plsc

_T = 16
_K = 3072
_N = 24000
_B = 4
_C = 6
_L = 16
_NV = _N // _L
_SEL = _K + 2 * _L
_SELV = _SEL // _L
_HBINS = 4096
_INF = 0x7F800000
_BIG = 0x7FFFFFFF


def _d2_body(x_ref, lab_ref, keys_ref):
    xb = x_ref[0]
    coords = jnp.transpose(xb[0:3, :], (1, 0))
    lab = lab_ref[0]

    tooth = lax.broadcasted_iota(jnp.int32, (_T, _N), 0)
    onehot = (lab == tooth).astype(jnp.float32)
    counts = jnp.sum(onehot, axis=1)
    sums = lax.dot_general(onehot, coords, (((1,), (0,)), ((), ())),
                           preferred_element_type=jnp.float32)
    cen = sums / jnp.maximum(counts, 1.0)[:, None]

    px = xb[0:1, :]
    py = xb[1:2, :]
    pz = xb[2:3, :]
    dx = cen[:, 0:1] - px
    dy = cen[:, 1:2] - py
    dz = cen[:, 2:3] - pz
    d2 = (dx * dx + dy * dy) + dz * dz
    keys_ref[0] = lax.bitcast_convert_type(d2, jnp.int32)


def _sc_body(keys_hbm, x_hbm, idx_hbm, cent_hbm,
             keys_v, selk_a, seli_a, selk_b, seli_b, tie_v,
             hist_v, pref_v, gbuf, obuf, sem):
    wid = lax.axis_index("s") * 2 + lax.axis_index("c")

    def process_row(j, _):
        row = wid + 32 * j
        b = row // _T

        pltpu.sync_copy(keys_hbm.at[row], keys_v)

        def init_body(i, c):
            selk_a[pl.ds(i * _L, _L)] = jnp.full((_L,), _INF, jnp.int32)
            seli_a[pl.ds(i * _L, _L)] = jnp.full((_L,), _BIG, jnp.int32)
            tie_v[pl.ds(i * _L, _L)] = jnp.full((_L,), _BIG, jnp.int32)
            return c
        lax.fori_loop(0, _SELV, init_body, 0)

        def clear12_body(i, c):
            hist_v[pl.ds(i * _L, _L)] = jnp.zeros((_L,), jnp.int32)
            return c
        lax.fori_loop(0, _HBINS // _L, clear12_body, 0)

        def hist12_body(v, c):
            kv = keys_v[pl.ds(v * _L, _L)]
            digit = lax.shift_right_logical(kv, 20)
            cnt, lm = plsc.scan_count(digit)
            plsc.addupdate_scatter(hist_v, [digit], cnt, mask=lm)
            return c
        lax.fori_loop(0, _NV, hist12_body, 0)

        def beta12_body(i, c):
            cum, beta1, nb1, m1 = c
            h = hist_v[pl.ds(i * _L, _L)]
            inc = plsc.cumsum(h) + cum
            bins = lax.iota(jnp.int32, _L) + i * _L
            cand = jnp.where(inc > _K - 1, bins, _HBINS)
            bmin = jnp.min(cand)
            hit = jnp.logical_and(beta1 == _HBINS, bmin < _HBINS)
            nb_here = cum + jnp.sum(jnp.where(bins < bmin, h, 0))
            m_here = jnp.sum(jnp.where(bins == bmin, h, 0))
            nb1 = jnp.where(hit, nb_here, nb1)
            m1 = jnp.where(hit, m_here, m1)
            return (cum + jnp.sum(h), jnp.minimum(beta1, bmin), nb1, m1)
        _, beta1, nb1, m1 = lax.fori_loop(
            0, _HBINS // _L, beta12_body,
            (jnp.int32(0), jnp.int32(_HBINS), jnp.int32(0), jnp.int32(0)))

        @pl.when(m1 <= _SEL - _L)
        def _():
            def fp_scan2(v, carry):
                off, coff = carry
                kv = keys_v[pl.ds(v * _L, _L)]
                idxv = lax.iota(jnp.int32, _L) + v * _L
                top12 = lax.shift_right_logical(kv, 20)
                m_lt = top12 < beta1
                plsc.store_compressed(selk_a.at[pl.ds(off, _L)], kv,
                                      mask=m_lt)
                plsc.store_compressed(seli_a.at[pl.ds(off, _L)], idxv,
                                      mask=m_lt)
                m_eq = top12 == beta1
                plsc.store_compressed(selk_b.at[pl.ds(coff, _L)], kv,
                                      mask=m_eq)
                plsc.store_compressed(seli_b.at[pl.ds(coff, _L)], idxv,
                                      mask=m_eq)
                return (off + jnp.sum(m_lt.astype(jnp.int32)),
                        coff + jnp.sum(m_eq.astype(jnp.int32)))
            lax.fori_loop(0, _NV, fp_scan2, (jnp.int32(0), jnp.int32(0)))

            mv = (m1 + _L - 1) // _L

            pref_full = beta1
            kth_c = jnp.int32(_K - 1) - nb1
            n_less = nb1
            for shift, nbits in ((12, 8), (4, 8), (0, 4)):
                nchunk = (1 << nbits) // _L

                def cl_body(i, c):
                    hist_v[pl.ds(i * _L, _L)] = jnp.zeros((_L,), jnp.int32)
                    return c
                lax.fori_loop(0, nchunk, cl_body, 0)

                def ch_body(v, c, _s=shift, _n=nbits, _p=pref_full, _m1=m1):
                    kv = selk_b[pl.ds(v * _L, _L)]
                    rows = lax.iota(jnp.int32, _L) + v * _L
                    m = jnp.logical_and(
                        rows < _m1,
                        lax.shift_right_logical(kv, _s + _n) == _p)
                    digit = lax.shift_right_logical(kv, _s) & ((1 << _n) - 1)
                    cnt, lm = plsc.scan_count(digit, mask=m)
                    plsc.addupdate_scatter(hist_v, [digit], cnt,
                                           mask=jnp.logical_and(lm, m))
                    return c
                lax.fori_loop(0, mv, ch_body, 0)

                def cb_body(i, c, _n=nbits, _k=kth_c):
                    cum, beta, nb = c
                    h = hist_v[pl.ds(i * _L, _L)]
                    inc = plsc.cumsum(h) + cum
                    bins = lax.iota(jnp.int32, _L) + i * _L
                    cand = jnp.where(inc > _k, bins, 1 << _n)
                    bmin = jnp.min(cand)
                    hit = jnp.logical_and(beta == (1 << _n), bmin < (1 << _n))
                    nb_here = cum + jnp.sum(jnp.where(bins < bmin, h, 0))
                    nb = jnp.where(hit, nb_here, nb)
                    return (cum + jnp.sum(h), jnp.minimum(beta, bmin), nb)
                _, beta_l, nb_l = lax.fori_loop(
                    0, nchunk, cb_body,
                    (jnp.int32(0), jnp.int32(1 << nbits), jnp.int32(0)))

                pref_full = pref_full * (1 << nbits) + beta_l
                kth_c = kth_c - nb_l
                n_less = n_less + nb_l

            thr = pref_full
            needed = kth_c + 1

            def fp_comp(v, carry, _m1=m1):
                off, toff = carry
                kv = selk_b[pl.ds(v * _L, _L)]
                iv = seli_b[pl.ds(v * _L, _L)]
                rows = lax.iota(jnp.int32, _L) + v * _L
                mval = rows < _m1
                m_lt = jnp.logical_and(kv < thr, mval)
                plsc.store_compressed(selk_a.at[pl.ds(off, _L)], kv,
                                      mask=m_lt)
                plsc.store_compressed(seli_a.at[pl.ds(off, _L)], iv,
                                      mask=m_lt)
                m_eq = jnp.logical_and(kv == thr, mval)

                @pl.when(toff < needed)
                def _():
                    plsc.store_compressed(tie_v.at[pl.ds(toff, _L)], iv,
                                          mask=m_eq)

                off = off + jnp.sum(m_lt.astype(jnp.int32))
                toff = jnp.where(toff < needed,
                                 toff + jnp.sum(m_eq.astype(jnp.int32)), toff)
                return off, toff
            lax.fori_loop(0, mv, fp_comp, (nb1, jnp.int32(0)))

            def fp_app(i, c):
                selk_a[pl.ds(n_less + i * _L, _L)] = (
                    jnp.full((_L,), 0, jnp.int32) + thr)
                seli_a[pl.ds(n_less + i * _L, _L)] = tie_v[pl.ds(i * _L, _L)]
                return c
            lax.fori_loop(0, (needed + _L - 1) // _L, fp_app, 0)

        @pl.when(m1 > _SEL - _L)
        def _():
            def select_level(lvl, carry):
                pref, kth, nl = carry
                shift = 24 - 8 * lvl

                def clear_body(i, c):
                    hist_v[pl.ds(i * _L, _L)] = jnp.zeros((_L,), jnp.int32)
                    return c
                lax.fori_loop(0, 256 // _L, clear_body, 0)

                def hist_body(v, c):
                    kv = keys_v[pl.ds(v * _L, _L)]
                    m = lax.shift_right_logical(kv, shift + 8) == pref
                    digit = lax.shift_right_logical(kv, shift) & 255
                    cnt, lm = plsc.scan_count(digit, mask=m)
                    plsc.addupdate_scatter(hist_v, [digit], cnt,
                                           mask=jnp.logical_and(lm, m))
                    return c
                lax.fori_loop(0, _NV, hist_body, 0)

                def beta_body(i, c):
                    cum, beta = c
                    h = hist_v[pl.ds(i * _L, _L)]
                    inc = plsc.cumsum(h) + cum
                    bins = lax.iota(jnp.int32, _L) + i * _L
                    cand = jnp.where(inc > kth, bins, 256)
                    return cum + jnp.sum(h), jnp.minimum(beta, jnp.min(cand))
                _, beta = lax.fori_loop(0, 256 // _L, beta_body,
                                        (jnp.int32(0), jnp.int32(256)))

                def nb_body(i, acc):
                    h = hist_v[pl.ds(i * _L, _L)]
                    bins = lax.iota(jnp.int32, _L) + i * _L
                    return acc + jnp.sum(jnp.where(bins < beta, h, 0))
                nb = lax.fori_loop(0, 256 // _L, nb_body, jnp.int32(0))

                return pref * 256 + beta, kth - nb, nl + nb

            pref, kth, n_less = lax.fori_loop(
                0, 4, select_level,
                (jnp.int32(0), jnp.int32(_K - 1), jnp.int32(0)))
            thr = pref
            needed = kth + 1

            def comp_body(v, carry):
                off, toff = carry
                kv = keys_v[pl.ds(v * _L, _L)]
                idxv = lax.iota(jnp.int32, _L) + v * _L
                m_lt = kv < thr
                plsc.store_compressed(selk_a.at[pl.ds(off, _L)], kv,
                                      mask=m_lt)
                plsc.store_compressed(seli_a.at[pl.ds(off, _L)], idxv,
                                      mask=m_lt)
                m_eq = kv == thr

                @pl.when(toff < needed)
                def _():
                    plsc.store_compressed(tie_v.at[pl.ds(toff, _L)], idxv,
                                          mask=m_eq)

                off = off + jnp.sum(m_lt.astype(jnp.int32))
                toff = jnp.where(toff < needed,
                                 toff + jnp.sum(m_eq.astype(jnp.int32)), toff)
                return off, toff
            lax.fori_loop(0, _NV, comp_body, (jnp.int32(0), jnp.int32(0)))

            def app_body(i, c):
                selk_a[pl.ds(n_less + i * _L, _L)] = (
                    jnp.full((_L,), 0, jnp.int32) + thr)
                seli_a[pl.ds(n_less + i * _L, _L)] = tie_v[pl.ds(i * _L, _L)]
                return c
            lax.fori_loop(0, (needed + _L - 1) // _L, app_body, 0)

        def sort_pass(p, sk, si, dk, di):
            shift = 8 * p

            def clear_body(i, c):
                hist_v[pl.ds(i * _L, _L)] = jnp.zeros((_L,), jnp.int32)
                return c
            lax.fori_loop(0, 256 // _L, clear_body, 0)

            def hist_body(v, c):
                kv = sk[pl.ds(v * _L, _L)]
                digit = lax.shift_right_logical(kv, shift) & 255
                cnt, lm = plsc.scan_count(digit)
                plsc.addupdate_scatter(hist_v, [digit], cnt, mask=lm)
                return c
            lax.fori_loop(0, _SELV, hist_body, 0)

            def pf_body(i, cum):
                h = hist_v[pl.ds(i * _L, _L)]
                inc = plsc.cumsum(h)
                pref_v[pl.ds(i * _L, _L)] = inc - h + cum
                return cum + jnp.sum(h)
            lax.fori_loop(0, 256 // _L, pf_body, jnp.int32(0))

            def perm_body(v, c):
                kv = sk[pl.ds(v * _L, _L)]
                iv = si[pl.ds(v * _L, _L)]
                digit = lax.shift_right_logical(kv, shift) & 255
                cnt, lm = plsc.scan_count(digit)
                base = plsc.load_gather(pref_v, [digit])
                pos = base + cnt - 1
                plsc.store_scatter(dk, [pos], kv)
                plsc.store_scatter(di, [pos], iv)
                plsc.addupdate_scatter(pref_v, [digit], cnt, mask=lm)
                return c
            lax.fori_loop(0, _SELV, perm_body, 0)

        sort_pass(0, selk_a, seli_a, selk_b, seli_b)
        sort_pass(1, selk_b, seli_b, selk_a, seli_a)
        sort_pass(2, selk_a, seli_a, selk_b, seli_b)
        sort_pass(3, selk_b, seli_b, selk_a, seli_a)

        pltpu.sync_copy(seli_a.at[pl.ds(0, _K)], idx_hbm.at[row])

        copies = [
            pltpu.async_copy(
                x_hbm.at[b].at[c].at[seli_a.at[pl.ds(0, _K)]],
                gbuf.at[c], sem)
            for c in range(_C)
        ]
        for cp in copies:
            cp.wait()

        for c in range(3):
            ch = gbuf.at[c]

            def acc_body(i, acc, _ch=ch):
                return acc + _ch[pl.ds(i * _L, _L)]
            acc = lax.fori_loop(0, _K // _L, acc_body,
                                jnp.zeros((_L,), jnp.float32))
            mean = jnp.sum(acc) * (1.0 / _K)

            def cen_body(i, cc, _ch=ch, _m=mean):
                obuf[pl.ds(i * _L, _L)] = _ch[pl.ds(i * _L, _L)] - _m
                return cc
            lax.fori_loop(0, _K // _L, cen_body, 0)

            pltpu.sync_copy(obuf, cent_hbm.at[row, c])

        for c in range(3, _C):
            pltpu.sync_copy(gbuf.at[c], cent_hbm.at[row, c])
        return 0

    lax.fori_loop(0, 2, process_row, 0)


@functools.partial(jax.jit, static_argnums=())
def _sc_stage(keys, x):
    return pl.kernel(
        _sc_body,
        out_type=[
            jax.ShapeDtypeStruct((_B * _T, _K), jnp.int32),
            jax.ShapeDtypeStruct((_B * _T, _C, _K), jnp.float32),
        ],
        mesh=plsc.VectorSubcoreMesh(core_axis_name="c", subcore_axis_name="s"),
        compiler_params=pltpu.CompilerParams(needs_layout_passes=False,
                                             use_tc_tiling_on_sc=False),
        scratch_types=[
            pltpu.VMEM((_N,), jnp.int32),
            pltpu.VMEM((_SEL,), jnp.int32),
            pltpu.VMEM((_SEL,), jnp.int32),
            pltpu.VMEM((_SEL,), jnp.int32),
            pltpu.VMEM((_SEL,), jnp.int32),
            pltpu.VMEM((_SEL,), jnp.int32),
            pltpu.VMEM((_HBINS,), jnp.int32),
            pltpu.VMEM((256,), jnp.int32),
            pltpu.VMEM((_C, _K), jnp.float32),
            pltpu.VMEM((_K,), jnp.float32),
            pltpu.SemaphoreType.DMA,
        ],
    )(keys, x)


def kernel(inputs_0, inputs_1):
    x = inputs_0
    labels = inputs_1
    B, C, N = x.shape

    keys = pl.pallas_call(
        _d2_body,
        grid=(B,),
        in_specs=[
            pl.BlockSpec((1, C, N), lambda i: (i, 0, 0)),
            pl.BlockSpec((1, 1, N), lambda i: (i, 0, 0)),
        ],
        out_specs=pl.BlockSpec((1, _T, N), lambda i: (i, 0, 0)),
        out_shape=jax.ShapeDtypeStruct((B, _T, N), jnp.int32),
    )(x, labels)
    keys = keys.reshape(B * _T, N)

    nn_flat, centered = _sc_stage(keys, x)
    return centered, nn_flat.reshape(B, _T, _K)


# speedup vs baseline: 1.9062x; 1.0272x over previous
import functools

import jax
import jax.numpy as jnp
from jax import lax
from jax.experimental import pallas as pl
from jax.experimental.pallas import tpu as pltpu
from jax.experimental.pallas import tpu_sc as plsc

_T = 16
_K = 3072
_N = 24000
_B = 4
_C = 6
_L = 16
_NV = _N // _L
_SEL = _K + 2 * _L
_SELV = _SEL // _L
_HBINS = 4096
_INF = 0x7F800000
_BIG = 0x7FFFFFFF


def _d2_body(x_ref, lab_ref, keys_ref):
    xb = x_ref[0]
    coords = jnp.transpose(xb[0:3, :], (1, 0))
    lab = lab_ref[0]

    tooth = lax.broadcasted_iota(jnp.int32, (_T, _N), 0)
    onehot = (lab == tooth).astype(jnp.float32)
    counts = jnp.sum(onehot, axis=1)
    sums = lax.dot_general(onehot, coords, (((1,), (0,)), ((), ())),
                           preferred_element_type=jnp.float32)
    cen = sums / jnp.maximum(counts, 1.0)[:, None]

    px = xb[0:1, :]
    py = xb[1:2, :]
    pz = xb[2:3, :]
    dx = cen[:, 0:1] - px
    dy = cen[:, 1:2] - py
    dz = cen[:, 2:3] - pz
    d2 = (dx * dx + dy * dy) + dz * dz
    keys_ref[0] = lax.bitcast_convert_type(d2, jnp.int32)


def _sc_body(keys_hbm, x_hbm, idx_hbm, cent_hbm,
             keys_v, selk_a, seli_a, selk_b, seli_b, tie_v,
             hist_v, pref_v, gbuf, obuf, sem):
    wid = lax.axis_index("s") * 2 + lax.axis_index("c")

    def process_row(j, _):
        row = wid + 32 * j
        b = row // _T

        pltpu.sync_copy(keys_hbm.at[row], keys_v)

        def init_body(i, c):
            selk_a[pl.ds(i * _L, _L)] = jnp.full((_L,), _INF, jnp.int32)
            seli_a[pl.ds(i * _L, _L)] = jnp.full((_L,), _BIG, jnp.int32)
            tie_v[pl.ds(i * _L, _L)] = jnp.full((_L,), _BIG, jnp.int32)
            return c
        lax.fori_loop(0, _SELV, init_body, 0)

        def clear12_body(i, c):
            hist_v[pl.ds(i * _L, _L)] = jnp.zeros((_L,), jnp.int32)
            return c
        lax.fori_loop(0, _HBINS // _L, clear12_body, 0)

        def hist12_body(v, c):
            kv = keys_v[pl.ds(v * _L, _L)]
            digit = lax.shift_right_logical(kv, 20)
            cnt, lm = plsc.scan_count(digit)
            plsc.addupdate_scatter(hist_v, [digit], cnt, mask=lm)
            return c
        lax.fori_loop(0, _NV, hist12_body, 0)

        def beta12_body(i, c):
            cum, beta1, nb1, m1 = c
            h = hist_v[pl.ds(i * _L, _L)]
            inc = plsc.cumsum(h) + cum
            bins = lax.iota(jnp.int32, _L) + i * _L
            cand = jnp.where(inc > _K - 1, bins, _HBINS)
            bmin = jnp.min(cand)
            hit = jnp.logical_and(beta1 == _HBINS, bmin < _HBINS)
            nb_here = cum + jnp.sum(jnp.where(bins < bmin, h, 0))
            m_here = jnp.sum(jnp.where(bins == bmin, h, 0))
            nb1 = jnp.where(hit, nb_here, nb1)
            m1 = jnp.where(hit, m_here, m1)
            return (cum + jnp.sum(h), jnp.minimum(beta1, bmin), nb1, m1)
        _, beta1, nb1, m1 = lax.fori_loop(
            0, _HBINS // _L, beta12_body,
            (jnp.int32(0), jnp.int32(_HBINS), jnp.int32(0), jnp.int32(0)))

        @pl.when(m1 <= _SEL - _L)
        def _():
            def fp_scan2(v, carry):
                off, coff = carry
                kv = keys_v[pl.ds(v * _L, _L)]
                idxv = lax.iota(jnp.int32, _L) + v * _L
                top12 = lax.shift_right_logical(kv, 20)
                m_lt = top12 < beta1
                plsc.store_compressed(selk_a.at[pl.ds(off, _L)], kv,
                                      mask=m_lt)
                plsc.store_compressed(seli_a.at[pl.ds(off, _L)], idxv,
                                      mask=m_lt)
                m_eq = top12 == beta1
                plsc.store_compressed(selk_b.at[pl.ds(coff, _L)], kv,
                                      mask=m_eq)
                plsc.store_compressed(seli_b.at[pl.ds(coff, _L)], idxv,
                                      mask=m_eq)
                return (off + jnp.sum(m_lt.astype(jnp.int32)),
                        coff + jnp.sum(m_eq.astype(jnp.int32)))
            lax.fori_loop(0, _NV, fp_scan2, (jnp.int32(0), jnp.int32(0)))

            mv = (m1 + _L - 1) // _L

            pref_full = beta1
            kth_c = jnp.int32(_K - 1) - nb1
            n_less = nb1
            for shift, nbits in ((12, 8), (4, 8), (0, 4)):
                nchunk = (1 << nbits) // _L

                def cl_body(i, c):
                    hist_v[pl.ds(i * _L, _L)] = jnp.zeros((_L,), jnp.int32)
                    return c
                lax.fori_loop(0, nchunk, cl_body, 0)

                def ch_body(v, c, _s=shift, _n=nbits, _p=pref_full, _m1=m1):
                    kv = selk_b[pl.ds(v * _L, _L)]
                    rows = lax.iota(jnp.int32, _L) + v * _L
                    m = jnp.logical_and(
                        rows < _m1,
                        lax.shift_right_logical(kv, _s + _n) == _p)
                    digit = lax.shift_right_logical(kv, _s) & ((1 << _n) - 1)
                    cnt, lm = plsc.scan_count(digit, mask=m)
                    plsc.addupdate_scatter(hist_v, [digit], cnt,
                                           mask=jnp.logical_and(lm, m))
                    return c
                lax.fori_loop(0, mv, ch_body, 0)

                def cb_body(i, c, _n=nbits, _k=kth_c):
                    cum, beta, nb = c
                    h = hist_v[pl.ds(i * _L, _L)]
                    inc = plsc.cumsum(h) + cum
                    bins = lax.iota(jnp.int32, _L) + i * _L
                    cand = jnp.where(inc > _k, bins, 1 << _n)
                    bmin = jnp.min(cand)
                    hit = jnp.logical_and(beta == (1 << _n), bmin < (1 << _n))
                    nb_here = cum + jnp.sum(jnp.where(bins < bmin, h, 0))
                    nb = jnp.where(hit, nb_here, nb)
                    return (cum + jnp.sum(h), jnp.minimum(beta, bmin), nb)
                _, beta_l, nb_l = lax.fori_loop(
                    0, nchunk, cb_body,
                    (jnp.int32(0), jnp.int32(1 << nbits), jnp.int32(0)))

                pref_full = pref_full * (1 << nbits) + beta_l
                kth_c = kth_c - nb_l
                n_less = n_less + nb_l

            thr = pref_full
            needed = kth_c + 1

            def fp_comp(v, carry, _m1=m1):
                off, toff = carry
                kv = selk_b[pl.ds(v * _L, _L)]
                iv = seli_b[pl.ds(v * _L, _L)]
                rows = lax.iota(jnp.int32, _L) + v * _L
                mval = rows < _m1
                m_lt = jnp.logical_and(kv < thr, mval)
                plsc.store_compressed(selk_a.at[pl.ds(off, _L)], kv,
                                      mask=m_lt)
                plsc.store_compressed(seli_a.at[pl.ds(off, _L)], iv,
                                      mask=m_lt)
                m_eq = jnp.logical_and(kv == thr, mval)

                @pl.when(toff < needed)
                def _():
                    plsc.store_compressed(tie_v.at[pl.ds(toff, _L)], iv,
                                          mask=m_eq)

                off = off + jnp.sum(m_lt.astype(jnp.int32))
                toff = jnp.where(toff < needed,
                                 toff + jnp.sum(m_eq.astype(jnp.int32)), toff)
                return off, toff
            lax.fori_loop(0, mv, fp_comp, (nb1, jnp.int32(0)))

            def fp_app(i, c):
                selk_a[pl.ds(n_less + i * _L, _L)] = (
                    jnp.full((_L,), 0, jnp.int32) + thr)
                seli_a[pl.ds(n_less + i * _L, _L)] = tie_v[pl.ds(i * _L, _L)]
                return c
            lax.fori_loop(0, (needed + _L - 1) // _L, fp_app, 0)

        @pl.when(m1 > _SEL - _L)
        def _():
            def select_level(lvl, carry):
                pref, kth, nl = carry
                shift = 24 - 8 * lvl

                def clear_body(i, c):
                    hist_v[pl.ds(i * _L, _L)] = jnp.zeros((_L,), jnp.int32)
                    return c
                lax.fori_loop(0, 256 // _L, clear_body, 0)

                def hist_body(v, c):
                    kv = keys_v[pl.ds(v * _L, _L)]
                    m = lax.shift_right_logical(kv, shift + 8) == pref
                    digit = lax.shift_right_logical(kv, shift) & 255
                    cnt, lm = plsc.scan_count(digit, mask=m)
                    plsc.addupdate_scatter(hist_v, [digit], cnt,
                                           mask=jnp.logical_and(lm, m))
                    return c
                lax.fori_loop(0, _NV, hist_body, 0)

                def beta_body(i, c):
                    cum, beta = c
                    h = hist_v[pl.ds(i * _L, _L)]
                    inc = plsc.cumsum(h) + cum
                    bins = lax.iota(jnp.int32, _L) + i * _L
                    cand = jnp.where(inc > kth, bins, 256)
                    return cum + jnp.sum(h), jnp.minimum(beta, jnp.min(cand))
                _, beta = lax.fori_loop(0, 256 // _L, beta_body,
                                        (jnp.int32(0), jnp.int32(256)))

                def nb_body(i, acc):
                    h = hist_v[pl.ds(i * _L, _L)]
                    bins = lax.iota(jnp.int32, _L) + i * _L
                    return acc + jnp.sum(jnp.where(bins < beta, h, 0))
                nb = lax.fori_loop(0, 256 // _L, nb_body, jnp.int32(0))

                return pref * 256 + beta, kth - nb, nl + nb

            pref, kth, n_less = lax.fori_loop(
                0, 4, select_level,
                (jnp.int32(0), jnp.int32(_K - 1), jnp.int32(0)))
            thr = pref
            needed = kth + 1

            def comp_body(v, carry):
                off, toff = carry
                kv = keys_v[pl.ds(v * _L, _L)]
                idxv = lax.iota(jnp.int32, _L) + v * _L
                m_lt = kv < thr
                plsc.store_compressed(selk_a.at[pl.ds(off, _L)], kv,
                                      mask=m_lt)
                plsc.store_compressed(seli_a.at[pl.ds(off, _L)], idxv,
                                      mask=m_lt)
                m_eq = kv == thr

                @pl.when(toff < needed)
                def _():
                    plsc.store_compressed(tie_v.at[pl.ds(toff, _L)], idxv,
                                          mask=m_eq)

                off = off + jnp.sum(m_lt.astype(jnp.int32))
                toff = jnp.where(toff < needed,
                                 toff + jnp.sum(m_eq.astype(jnp.int32)), toff)
                return off, toff
            lax.fori_loop(0, _NV, comp_body, (jnp.int32(0), jnp.int32(0)))

            def app_body(i, c):
                selk_a[pl.ds(n_less + i * _L, _L)] = (
                    jnp.full((_L,), 0, jnp.int32) + thr)
                seli_a[pl.ds(n_less + i * _L, _L)] = tie_v[pl.ds(i * _L, _L)]
                return c
            lax.fori_loop(0, (needed + _L - 1) // _L, app_body, 0)

        def sort_pass(p, sk, si, dk, di):
            shift = 8 * p

            def clear_body(i, c):
                hist_v[pl.ds(i * _L, _L)] = jnp.zeros((_L,), jnp.int32)
                return c
            lax.fori_loop(0, 256 // _L, clear_body, 0)

            def hist_body(v, c):
                kv = sk[pl.ds(v * _L, _L)]
                digit = lax.shift_right_logical(kv, shift) & 255
                cnt, lm = plsc.scan_count(digit)
                plsc.addupdate_scatter(hist_v, [digit], cnt, mask=lm)
                return c
            lax.fori_loop(0, _SELV, hist_body, 0)

            def pf_body(i, cum):
                h = hist_v[pl.ds(i * _L, _L)]
                inc = plsc.cumsum(h)
                pref_v[pl.ds(i * _L, _L)] = inc - h + cum
                return cum + jnp.sum(h)
            lax.fori_loop(0, 256 // _L, pf_body, jnp.int32(0))

            def perm_body(v, c):
                kv = sk[pl.ds(v * _L, _L)]
                iv = si[pl.ds(v * _L, _L)]
                digit = lax.shift_right_logical(kv, shift) & 255
                cnt, lm = plsc.scan_count(digit)
                base = plsc.load_gather(pref_v, [digit])
                pos = base + cnt - 1
                plsc.store_scatter(dk, [pos], kv)
                plsc.store_scatter(di, [pos], iv)
                plsc.addupdate_scatter(pref_v, [digit], cnt, mask=lm)
                return c
            lax.fori_loop(0, _SELV, perm_body, 0)

        sort_pass(0, selk_a, seli_a, selk_b, seli_b)
        sort_pass(1, selk_b, seli_b, selk_a, seli_a)
        sort_pass(2, selk_a, seli_a, selk_b, seli_b)
        sort_pass(3, selk_b, seli_b, selk_a, seli_a)

        idx_cp = pltpu.async_copy(seli_a.at[pl.ds(0, _K)], idx_hbm.at[row],
                                  sem)

        copies = [
            pltpu.async_copy(
                x_hbm.at[b].at[c].at[seli_a.at[pl.ds(0, _K)]],
                gbuf.at[c], sem)
            for c in range(_C)
        ]
        out_cps = []

        for c in range(3):
            copies[c].wait()
            ch = gbuf.at[c]
            ob = obuf.at[c]

            def acc_body(i, acc, _ch=ch):
                return acc + _ch[pl.ds(i * _L, _L)]
            acc = lax.fori_loop(0, _K // _L, acc_body,
                                jnp.zeros((_L,), jnp.float32))
            mean = jnp.sum(acc) * (1.0 / _K)

            def cen_body(i, cc, _ch=ch, _ob=ob, _m=mean):
                _ob[pl.ds(i * _L, _L)] = _ch[pl.ds(i * _L, _L)] - _m
                return cc
            lax.fori_loop(0, _K // _L, cen_body, 0)

            out_cps.append(pltpu.async_copy(ob, cent_hbm.at[row, c], sem))

        for c in range(3, _C):
            copies[c].wait()
            out_cps.append(pltpu.async_copy(gbuf.at[c], cent_hbm.at[row, c],
                                            sem))

        idx_cp.wait()
        for cp in out_cps:
            cp.wait()
        return 0

    lax.fori_loop(0, 2, process_row, 0)


@functools.partial(jax.jit, static_argnums=())
def _sc_stage(keys, x):
    return pl.kernel(
        _sc_body,
        out_type=[
            jax.ShapeDtypeStruct((_B * _T, _K), jnp.int32),
            jax.ShapeDtypeStruct((_B * _T, _C, _K), jnp.float32),
        ],
        mesh=plsc.VectorSubcoreMesh(core_axis_name="c", subcore_axis_name="s"),
        compiler_params=pltpu.CompilerParams(needs_layout_passes=False,
                                             use_tc_tiling_on_sc=False),
        scratch_types=[
            pltpu.VMEM((_N,), jnp.int32),
            pltpu.VMEM((_SEL,), jnp.int32),
            pltpu.VMEM((_SEL,), jnp.int32),
            pltpu.VMEM((_SEL,), jnp.int32),
            pltpu.VMEM((_SEL,), jnp.int32),
            pltpu.VMEM((_SEL,), jnp.int32),
            pltpu.VMEM((_HBINS,), jnp.int32),
            pltpu.VMEM((256,), jnp.int32),
            pltpu.VMEM((_C, _K), jnp.float32),
            pltpu.VMEM((3, _K), jnp.float32),
            pltpu.SemaphoreType.DMA,
        ],
    )(keys, x)


def kernel(inputs_0, inputs_1):
    x = inputs_0
    labels = inputs_1
    B, C, N = x.shape

    keys = pl.pallas_call(
        _d2_body,
        grid=(B,),
        in_specs=[
            pl.BlockSpec((1, C, N), lambda i: (i, 0, 0)),
            pl.BlockSpec((1, 1, N), lambda i: (i, 0, 0)),
        ],
        out_specs=pl.BlockSpec((1, _T, N), lambda i: (i, 0, 0)),
        out_shape=jax.ShapeDtypeStruct((B, _T, N), jnp.int32),
    )(x, labels)
    keys = keys.reshape(B * _T, N)

    nn_flat, centered = _sc_stage(keys, x)
    return centered, nn_flat.reshape(B, _T, _K)


# speedup vs baseline: 1.9507x; 1.0234x over previous
import functools

import jax
import jax.numpy as jnp
from jax import lax
from jax.experimental import pallas as pl
from jax.experimental.pallas import tpu as pltpu
from jax.experimental.pallas import tpu_sc as plsc

_T = 16
_K = 3072
_N = 24000
_B = 4
_C = 6
_L = 16
_NV = _N // _L
_SEL = _K + 2 * _L
_SELV = _SEL // _L
_HBINS = 4096
_INF = 0x7F800000
_BIG = 0x7FFFFFFF


def _d2_body(x_ref, lab_ref, keys_ref):
    xb = x_ref[0]
    coords = jnp.transpose(xb[0:3, :], (1, 0))
    lab = lab_ref[0]

    tooth = lax.broadcasted_iota(jnp.int32, (_T, _N), 0)
    onehot = (lab == tooth).astype(jnp.float32)
    counts = jnp.sum(onehot, axis=1)
    sums = lax.dot_general(onehot, coords, (((1,), (0,)), ((), ())),
                           preferred_element_type=jnp.float32)
    cen = sums / jnp.maximum(counts, 1.0)[:, None]

    px = xb[0:1, :]
    py = xb[1:2, :]
    pz = xb[2:3, :]
    dx = cen[:, 0:1] - px
    dy = cen[:, 1:2] - py
    dz = cen[:, 2:3] - pz
    d2 = (dx * dx + dy * dy) + dz * dz
    keys_ref[0] = lax.bitcast_convert_type(d2, jnp.int32)


def _sc_body(keys_hbm, x_hbm, idx_hbm, cent_hbm,
             keys_v, selk_a, seli_a, selk_b, seli_b, tie_v,
             hist_v, pref_v, gbuf, obuf, sem, gsem):
    wid = lax.axis_index("s") * 2 + lax.axis_index("c")

    def process_row(j, _):
        row = wid + 32 * j
        b = row // _T

        pltpu.sync_copy(keys_hbm.at[row], keys_v)

        def init_body(i, c):
            selk_a[pl.ds(i * _L, _L)] = jnp.full((_L,), _INF, jnp.int32)
            seli_a[pl.ds(i * _L, _L)] = jnp.full((_L,), _BIG, jnp.int32)
            tie_v[pl.ds(i * _L, _L)] = jnp.full((_L,), _BIG, jnp.int32)
            return c
        lax.fori_loop(0, _SELV, init_body, 0)

        def clear12_body(i, c):
            hist_v[pl.ds(i * _L, _L)] = jnp.zeros((_L,), jnp.int32)
            return c
        lax.fori_loop(0, _HBINS // _L, clear12_body, 0)

        def hist12_body(v, c):
            kv = keys_v[pl.ds(v * _L, _L)]
            digit = lax.shift_right_logical(kv, 20)
            cnt, lm = plsc.scan_count(digit)
            plsc.addupdate_scatter(hist_v, [digit], cnt, mask=lm)
            return c
        lax.fori_loop(0, _NV, hist12_body, 0)

        def beta12_body(i, c):
            cum, beta1, nb1, m1 = c
            h = hist_v[pl.ds(i * _L, _L)]
            inc = plsc.cumsum(h) + cum
            bins = lax.iota(jnp.int32, _L) + i * _L
            cand = jnp.where(inc > _K - 1, bins, _HBINS)
            bmin = jnp.min(cand)
            hit = jnp.logical_and(beta1 == _HBINS, bmin < _HBINS)
            nb_here = cum + jnp.sum(jnp.where(bins < bmin, h, 0))
            m_here = jnp.sum(jnp.where(bins == bmin, h, 0))
            nb1 = jnp.where(hit, nb_here, nb1)
            m1 = jnp.where(hit, m_here, m1)
            return (cum + jnp.sum(h), jnp.minimum(beta1, bmin), nb1, m1)
        _, beta1, nb1, m1 = lax.fori_loop(
            0, _HBINS // _L, beta12_body,
            (jnp.int32(0), jnp.int32(_HBINS), jnp.int32(0), jnp.int32(0)))

        @pl.when(m1 <= _SEL - _L)
        def _():
            def fp_scan2(v, carry):
                off, coff = carry
                kv = keys_v[pl.ds(v * _L, _L)]
                idxv = lax.iota(jnp.int32, _L) + v * _L
                top12 = lax.shift_right_logical(kv, 20)
                m_lt = top12 < beta1
                plsc.store_compressed(selk_a.at[pl.ds(off, _L)], kv,
                                      mask=m_lt)
                plsc.store_compressed(seli_a.at[pl.ds(off, _L)], idxv,
                                      mask=m_lt)
                m_eq = top12 == beta1
                plsc.store_compressed(selk_b.at[pl.ds(coff, _L)], kv,
                                      mask=m_eq)
                plsc.store_compressed(seli_b.at[pl.ds(coff, _L)], idxv,
                                      mask=m_eq)
                return (off + jnp.sum(m_lt.astype(jnp.int32)),
                        coff + jnp.sum(m_eq.astype(jnp.int32)))
            lax.fori_loop(0, _NV, fp_scan2, (jnp.int32(0), jnp.int32(0)))

            mv = (m1 + _L - 1) // _L

            pref_full = beta1
            kth_c = jnp.int32(_K - 1) - nb1
            n_less = nb1
            for shift, nbits in ((12, 8), (4, 8), (0, 4)):
                nchunk = (1 << nbits) // _L

                def cl_body(i, c):
                    hist_v[pl.ds(i * _L, _L)] = jnp.zeros((_L,), jnp.int32)
                    return c
                lax.fori_loop(0, nchunk, cl_body, 0)

                def ch_body(v, c, _s=shift, _n=nbits, _p=pref_full, _m1=m1):
                    kv = selk_b[pl.ds(v * _L, _L)]
                    rows = lax.iota(jnp.int32, _L) + v * _L
                    m = jnp.logical_and(
                        rows < _m1,
                        lax.shift_right_logical(kv, _s + _n) == _p)
                    digit = lax.shift_right_logical(kv, _s) & ((1 << _n) - 1)
                    cnt, lm = plsc.scan_count(digit, mask=m)
                    plsc.addupdate_scatter(hist_v, [digit], cnt,
                                           mask=jnp.logical_and(lm, m))
                    return c
                lax.fori_loop(0, mv, ch_body, 0)

                def cb_body(i, c, _n=nbits, _k=kth_c):
                    cum, beta, nb = c
                    h = hist_v[pl.ds(i * _L, _L)]
                    inc = plsc.cumsum(h) + cum
                    bins = lax.iota(jnp.int32, _L) + i * _L
                    cand = jnp.where(inc > _k, bins, 1 << _n)
                    bmin = jnp.min(cand)
                    hit = jnp.logical_and(beta == (1 << _n), bmin < (1 << _n))
                    nb_here = cum + jnp.sum(jnp.where(bins < bmin, h, 0))
                    nb = jnp.where(hit, nb_here, nb)
                    return (cum + jnp.sum(h), jnp.minimum(beta, bmin), nb)
                _, beta_l, nb_l = lax.fori_loop(
                    0, nchunk, cb_body,
                    (jnp.int32(0), jnp.int32(1 << nbits), jnp.int32(0)))

                pref_full = pref_full * (1 << nbits) + beta_l
                kth_c = kth_c - nb_l
                n_less = n_less + nb_l

            thr = pref_full
            needed = kth_c + 1

            def fp_comp(v, carry, _m1=m1):
                off, toff = carry
                kv = selk_b[pl.ds(v * _L, _L)]
                iv = seli_b[pl.ds(v * _L, _L)]
                rows = lax.iota(jnp.int32, _L) + v * _L
                mval = rows < _m1
                m_lt = jnp.logical_and(kv < thr, mval)
                plsc.store_compressed(selk_a.at[pl.ds(off, _L)], kv,
                                      mask=m_lt)
                plsc.store_compressed(seli_a.at[pl.ds(off, _L)], iv,
                                      mask=m_lt)
                m_eq = jnp.logical_and(kv == thr, mval)

                @pl.when(toff < needed)
                def _():
                    plsc.store_compressed(tie_v.at[pl.ds(toff, _L)], iv,
                                          mask=m_eq)

                off = off + jnp.sum(m_lt.astype(jnp.int32))
                toff = jnp.where(toff < needed,
                                 toff + jnp.sum(m_eq.astype(jnp.int32)), toff)
                return off, toff
            lax.fori_loop(0, mv, fp_comp, (nb1, jnp.int32(0)))

            def fp_app(i, c):
                selk_a[pl.ds(n_less + i * _L, _L)] = (
                    jnp.full((_L,), 0, jnp.int32) + thr)
                seli_a[pl.ds(n_less + i * _L, _L)] = tie_v[pl.ds(i * _L, _L)]
                return c
            lax.fori_loop(0, (needed + _L - 1) // _L, fp_app, 0)

        @pl.when(m1 > _SEL - _L)
        def _():
            def select_level(lvl, carry):
                pref, kth, nl = carry
                shift = 24 - 8 * lvl

                def clear_body(i, c):
                    hist_v[pl.ds(i * _L, _L)] = jnp.zeros((_L,), jnp.int32)
                    return c
                lax.fori_loop(0, 256 // _L, clear_body, 0)

                def hist_body(v, c):
                    kv = keys_v[pl.ds(v * _L, _L)]
                    m = lax.shift_right_logical(kv, shift + 8) == pref
                    digit = lax.shift_right_logical(kv, shift) & 255
                    cnt, lm = plsc.scan_count(digit, mask=m)
                    plsc.addupdate_scatter(hist_v, [digit], cnt,
                                           mask=jnp.logical_and(lm, m))
                    return c
                lax.fori_loop(0, _NV, hist_body, 0)

                def beta_body(i, c):
                    cum, beta = c
                    h = hist_v[pl.ds(i * _L, _L)]
                    inc = plsc.cumsum(h) + cum
                    bins = lax.iota(jnp.int32, _L) + i * _L
                    cand = jnp.where(inc > kth, bins, 256)
                    return cum + jnp.sum(h), jnp.minimum(beta, jnp.min(cand))
                _, beta = lax.fori_loop(0, 256 // _L, beta_body,
                                        (jnp.int32(0), jnp.int32(256)))

                def nb_body(i, acc):
                    h = hist_v[pl.ds(i * _L, _L)]
                    bins = lax.iota(jnp.int32, _L) + i * _L
                    return acc + jnp.sum(jnp.where(bins < beta, h, 0))
                nb = lax.fori_loop(0, 256 // _L, nb_body, jnp.int32(0))

                return pref * 256 + beta, kth - nb, nl + nb

            pref, kth, n_less = lax.fori_loop(
                0, 4, select_level,
                (jnp.int32(0), jnp.int32(_K - 1), jnp.int32(0)))
            thr = pref
            needed = kth + 1

            def comp_body(v, carry):
                off, toff = carry
                kv = keys_v[pl.ds(v * _L, _L)]
                idxv = lax.iota(jnp.int32, _L) + v * _L
                m_lt = kv < thr
                plsc.store_compressed(selk_a.at[pl.ds(off, _L)], kv,
                                      mask=m_lt)
                plsc.store_compressed(seli_a.at[pl.ds(off, _L)], idxv,
                                      mask=m_lt)
                m_eq = kv == thr

                @pl.when(toff < needed)
                def _():
                    plsc.store_compressed(tie_v.at[pl.ds(toff, _L)], idxv,
                                          mask=m_eq)

                off = off + jnp.sum(m_lt.astype(jnp.int32))
                toff = jnp.where(toff < needed,
                                 toff + jnp.sum(m_eq.astype(jnp.int32)), toff)
                return off, toff
            lax.fori_loop(0, _NV, comp_body, (jnp.int32(0), jnp.int32(0)))

            def app_body(i, c):
                selk_a[pl.ds(n_less + i * _L, _L)] = (
                    jnp.full((_L,), 0, jnp.int32) + thr)
                seli_a[pl.ds(n_less + i * _L, _L)] = tie_v[pl.ds(i * _L, _L)]
                return c
            lax.fori_loop(0, (needed + _L - 1) // _L, app_body, 0)

        def sort_pass(p, sk, si, dk, di):
            shift = 8 * p

            def clear_body(i, c):
                hist_v[pl.ds(i * _L, _L)] = jnp.zeros((_L,), jnp.int32)
                return c
            lax.fori_loop(0, 256 // _L, clear_body, 0)

            def hist_body(v, c):
                kv = sk[pl.ds(v * _L, _L)]
                digit = lax.shift_right_logical(kv, shift) & 255
                cnt, lm = plsc.scan_count(digit)
                plsc.addupdate_scatter(hist_v, [digit], cnt, mask=lm)
                return c
            lax.fori_loop(0, _SELV, hist_body, 0)

            def pf_body(i, cum):
                h = hist_v[pl.ds(i * _L, _L)]
                inc = plsc.cumsum(h)
                pref_v[pl.ds(i * _L, _L)] = inc - h + cum
                return cum + jnp.sum(h)
            lax.fori_loop(0, 256 // _L, pf_body, jnp.int32(0))

            def perm_body(v, c):
                kv = sk[pl.ds(v * _L, _L)]
                iv = si[pl.ds(v * _L, _L)]
                digit = lax.shift_right_logical(kv, shift) & 255
                cnt, lm = plsc.scan_count(digit)
                base = plsc.load_gather(pref_v, [digit])
                pos = base + cnt - 1
                plsc.store_scatter(dk, [pos], kv)
                plsc.store_scatter(di, [pos], iv)
                plsc.addupdate_scatter(pref_v, [digit], cnt, mask=lm)
                return c
            lax.fori_loop(0, _SELV, perm_body, 0)

        sort_pass(0, selk_a, seli_a, selk_b, seli_b)
        sort_pass(1, selk_b, seli_b, selk_a, seli_a)
        sort_pass(2, selk_a, seli_a, selk_b, seli_b)
        sort_pass(3, selk_b, seli_b, selk_a, seli_a)

        idx_cp = pltpu.async_copy(seli_a.at[pl.ds(0, _K)], idx_hbm.at[row],
                                  sem)

        copies = [
            pltpu.async_copy(
                x_hbm.at[b].at[c].at[seli_a.at[pl.ds(0, _K)]],
                gbuf.at[c], gsem.at[c])
            for c in range(_C)
        ]
        out_cps = []

        for c in range(3):
            copies[c].wait()
            ch = gbuf.at[c]
            ob = obuf.at[c]

            def acc_body(i, acc, _ch=ch):
                return acc + _ch[pl.ds(i * _L, _L)]
            acc = lax.fori_loop(0, _K // _L, acc_body,
                                jnp.zeros((_L,), jnp.float32))
            mean = jnp.sum(acc) * (1.0 / _K)

            def cen_body(i, cc, _ch=ch, _ob=ob, _m=mean):
                _ob[pl.ds(i * _L, _L)] = _ch[pl.ds(i * _L, _L)] - _m
                return cc
            lax.fori_loop(0, _K // _L, cen_body, 0)

            out_cps.append(pltpu.async_copy(ob, cent_hbm.at[row, c], sem))

        for c in range(3, _C):
            copies[c].wait()
            out_cps.append(pltpu.async_copy(gbuf.at[c], cent_hbm.at[row, c],
                                            sem))

        idx_cp.wait()
        for cp in out_cps:
            cp.wait()
        return 0

    lax.fori_loop(0, 2, process_row, 0)


@functools.partial(jax.jit, static_argnums=())
def _sc_stage(keys, x):
    return pl.kernel(
        _sc_body,
        out_type=[
            jax.ShapeDtypeStruct((_B * _T, _K), jnp.int32),
            jax.ShapeDtypeStruct((_B * _T, _C, _K), jnp.float32),
        ],
        mesh=plsc.VectorSubcoreMesh(core_axis_name="c", subcore_axis_name="s"),
        compiler_params=pltpu.CompilerParams(needs_layout_passes=False,
                                             use_tc_tiling_on_sc=False),
        scratch_types=[
            pltpu.VMEM((_N,), jnp.int32),
            pltpu.VMEM((_SEL,), jnp.int32),
            pltpu.VMEM((_SEL,), jnp.int32),
            pltpu.VMEM((_SEL,), jnp.int32),
            pltpu.VMEM((_SEL,), jnp.int32),
            pltpu.VMEM((_SEL,), jnp.int32),
            pltpu.VMEM((_HBINS,), jnp.int32),
            pltpu.VMEM((256,), jnp.int32),
            pltpu.VMEM((_C, _K), jnp.float32),
            pltpu.VMEM((3, _K), jnp.float32),
            pltpu.SemaphoreType.DMA,
            pltpu.SemaphoreType.DMA((_C,)),
        ],
    )(keys, x)


def kernel(inputs_0, inputs_1):
    x = inputs_0
    labels = inputs_1
    B, C, N = x.shape

    keys = pl.pallas_call(
        _d2_body,
        grid=(B,),
        in_specs=[
            pl.BlockSpec((1, C, N), lambda i: (i, 0, 0)),
            pl.BlockSpec((1, 1, N), lambda i: (i, 0, 0)),
        ],
        out_specs=pl.BlockSpec((1, _T, N), lambda i: (i, 0, 0)),
        out_shape=jax.ShapeDtypeStruct((B, _T, N), jnp.int32),
    )(x, labels)
    keys = keys.reshape(B * _T, N)

    nn_flat, centered = _sc_stage(keys, x)
    return centered, nn_flat.reshape(B, _T, _K)
